# baseline (speedup 1.0000x reference)
import jax
import jax.numpy as jnp
from jax import lax
from jax.experimental import pallas as pl
from jax.experimental.pallas import tpu as pltpu
from jax.experimental.pallas import tpu_sc as plsc

N = 10000
E = 320000
D = 128
H = 128
G = 64
STEPS = 4

NC = 2
NS = 16
CH = 64
NP = 10240
RPT = NP // NS
EPT = 10240
E_PAD = EPT * NC * NS
NCHUNK = EPT // CH


_sc_mesh = plsc.VectorSubcoreMesh(core_axis_name="c", subcore_axis_name="s")


NB = 2


def _sc_scatter_body(src_hbm, dst_hbm, m_hbm, zeros_hbm, out_hbm,
                     srcb, dstb, rows, agg, gsems):
    cid = lax.axis_index("c")
    sid = lax.axis_index("s")
    tid = cid * NS + sid
    pltpu.sync_copy(src_hbm.at[pl.ds(tid * EPT, EPT)], srcb)
    pltpu.sync_copy(dst_hbm.at[tid], dstb)
    pltpu.sync_copy(zeros_hbm, agg.at[pl.ds(sid * RPT, RPT)])
    plsc.subcore_barrier()

    for b in range(NB):
        pltpu.async_copy(m_hbm.at[srcb.at[pl.ds(b * CH, CH)]], rows.at[b],
                         gsems.at[b])

    def body(i, carry):
        j0 = i * NB
        for b in range(NB):
            j = j0 + b
            pltpu.make_async_copy(m_hbm.at[srcb.at[pl.ds(j * CH, CH)]],
                                  rows.at[b], gsems.at[b]).wait()
            pltpu.sync_copy(rows.at[b], agg.at[dstb.at[j]], add=True)

            @pl.when(j + NB < NCHUNK)
            def _():
                pltpu.async_copy(
                    m_hbm.at[srcb.at[pl.ds((j + NB) * CH, CH)]],
                    rows.at[b], gsems.at[b])
        return carry

    lax.fori_loop(0, NCHUNK // NB, body, 0)
    plsc.subcore_barrier()
    pltpu.sync_copy(agg.at[pl.ds(sid * RPT, RPT)],
                    out_hbm.at[pl.ds(cid * NP + sid * RPT, RPT)])


_sc_scatter = pl.kernel(
    _sc_scatter_body,
    out_type=jax.ShapeDtypeStruct((2 * NP, H), jnp.float32),
    mesh=_sc_mesh,
    scratch_types=[
        pltpu.VMEM((EPT,), jnp.int32),
        pltpu.VMEM((NCHUNK, CH), jnp.int32),
        pltpu.VMEM((NB, CH, H), jnp.float32),
        pltpu.VMEM_SHARED((NP, H), jnp.float32),
        pltpu.SemaphoreType.DMA((NB,)),
    ],
)


_BLK = 1000
_NBLK = N // _BLK


def _dense0_body(x_ref, w1_ref, b1_ref, wg0_ref, xp_ref, m0_ref):
    xp = jnp.dot(x_ref[...], w1_ref[...], preferred_element_type=jnp.float32)
    xp = jnp.maximum(xp + b1_ref[...], 0.0)
    xp_ref[...] = xp
    m0_ref[...] = jnp.dot(xp, wg0_ref[...], preferred_element_type=jnp.float32)


_dense0 = pl.pallas_call(
    _dense0_body,
    grid=(_NBLK,),
    in_specs=[
        pl.BlockSpec((_BLK, D), lambda i: (i, 0)),
        pl.BlockSpec((D, H), lambda i: (0, 0)),
        pl.BlockSpec((1, H), lambda i: (0, 0)),
        pl.BlockSpec((H, H), lambda i: (0, 0)),
    ],
    out_specs=[
        pl.BlockSpec((_BLK, H), lambda i: (i, 0)),
        pl.BlockSpec((_BLK, H), lambda i: (i, 0)),
    ],
    out_shape=[jax.ShapeDtypeStruct((N, H), jnp.float32),
               jax.ShapeDtypeStruct((N, H), jnp.float32)],
)


def _gru_core(agg, h, wih_ref, whh_ref, bih_ref, bhh_ref):
    gi = jnp.dot(agg, wih_ref[...], preferred_element_type=jnp.float32)
    gi = gi + bih_ref[...]
    gh = jnp.dot(h, whh_ref[...], preferred_element_type=jnp.float32)
    gh = gh + bhh_ref[...]
    r = jax.nn.sigmoid(gi[:, :H] + gh[:, :H])
    z = jax.nn.sigmoid(gi[:, H:2 * H] + gh[:, H:2 * H])
    n = jnp.tanh(gi[:, 2 * H:] + r * gh[:, 2 * H:])
    return (1.0 - z) * n + z * h


def _gru_step_body(agg_ref, h_ref, wih_ref, whh_ref, bih_ref, bhh_ref,
                   wgn_ref, h_out_ref, m_out_ref):
    agg = agg_ref[0] + agg_ref[1]
    hn = _gru_core(agg, h_ref[...], wih_ref, whh_ref, bih_ref, bhh_ref)
    h_out_ref[...] = hn
    m_out_ref[...] = jnp.dot(hn, wgn_ref[...], preferred_element_type=jnp.float32)


_gru_step = pl.pallas_call(
    _gru_step_body,
    grid=(_NBLK,),
    in_specs=[
        pl.BlockSpec((2, _BLK, H), lambda i: (0, i, 0)),
        pl.BlockSpec((_BLK, H), lambda i: (i, 0)),
        pl.BlockSpec((H, 3 * H), lambda i: (0, 0)),
        pl.BlockSpec((H, 3 * H), lambda i: (0, 0)),
        pl.BlockSpec((1, 3 * H), lambda i: (0, 0)),
        pl.BlockSpec((1, 3 * H), lambda i: (0, 0)),
        pl.BlockSpec((H, H), lambda i: (0, 0)),
    ],
    out_specs=[
        pl.BlockSpec((_BLK, H), lambda i: (i, 0)),
        pl.BlockSpec((_BLK, H), lambda i: (i, 0)),
    ],
    out_shape=[jax.ShapeDtypeStruct((N, H), jnp.float32),
               jax.ShapeDtypeStruct((N, H), jnp.float32)],
)


def _gru_last_body(agg_ref, h_ref, wih_ref, whh_ref, bih_ref, bhh_ref,
                   xp_ref, s2_ref, t2_ref, z_out_ref):
    agg = agg_ref[0] + agg_ref[1]
    hn = _gru_core(agg, h_ref[...], wih_ref, whh_ref, bih_ref, bhh_ref)
    z_out_ref[...] = jnp.maximum(
        xp_ref[...] + hn * s2_ref[...] + t2_ref[...], 0.0)


_gru_last = pl.pallas_call(
    _gru_last_body,
    grid=(_NBLK,),
    in_specs=[
        pl.BlockSpec((2, _BLK, H), lambda i: (0, i, 0)),
        pl.BlockSpec((_BLK, H), lambda i: (i, 0)),
        pl.BlockSpec((H, 3 * H), lambda i: (0, 0)),
        pl.BlockSpec((H, 3 * H), lambda i: (0, 0)),
        pl.BlockSpec((1, 3 * H), lambda i: (0, 0)),
        pl.BlockSpec((1, 3 * H), lambda i: (0, 0)),
        pl.BlockSpec((_BLK, H), lambda i: (i, 0)),
        pl.BlockSpec((1, H), lambda i: (0, 0)),
        pl.BlockSpec((1, H), lambda i: (0, 0)),
    ],
    out_specs=pl.BlockSpec((_BLK, H), lambda i: (i, 0)),
    out_shape=jax.ShapeDtypeStruct((N, H), jnp.float32),
)


def _pool_fc_body(z_ref, b_ref, wf1_ref, bf1_ref, wf2_ref, bf2_ref,
                  out_ref, pooled):
    zval = z_ref[...]
    bval = b_ref[...]
    neg = jnp.finfo(jnp.float32).min

    def body(g, carry):
        mg = jnp.where(bval == g, zval, neg)
        pooled[pl.ds(g, 1), :] = jnp.max(mg, axis=0, keepdims=True)
        return carry

    lax.fori_loop(0, G, body, 0)
    y = jnp.dot(pooled[...], wf1_ref[...], preferred_element_type=jnp.float32)
    y = jnp.maximum(y + bf1_ref[...], 0.0)
    out_ref[...] = jnp.dot(y, wf2_ref[...],
                           preferred_element_type=jnp.float32) + bf2_ref[...]


_pool_fc = pl.pallas_call(
    _pool_fc_body,
    in_specs=[
        pl.BlockSpec((N, H), lambda: (0, 0)),
        pl.BlockSpec((N, 1), lambda: (0, 0)),
        pl.BlockSpec((H, H // 2), lambda: (0, 0)),
        pl.BlockSpec((1, H // 2), lambda: (0, 0)),
        pl.BlockSpec((H // 2, 2), lambda: (0, 0)),
        pl.BlockSpec((1, 2), lambda: (0, 0)),
    ],
    out_specs=pl.BlockSpec((G, 2), lambda: (0, 0)),
    out_shape=jax.ShapeDtypeStruct((G, 2), jnp.float32),
    scratch_shapes=[pltpu.VMEM((G, H), jnp.float32)],
)


def kernel(x, edge_index, batch, W_in, b_in, bn1_g, bn1_b, bn1_m, bn1_v, Wg,
           W_ih, W_hh, b_ih, b_hh, bn2_g, bn2_b, bn2_m, bn2_v, W_fc1, b_fc1,
           bn3_g, bn3_b, bn3_m, bn3_v, W_fc2, b_fc2):
    s1 = bn1_g / jnp.sqrt(bn1_v + 1e-5)
    W1 = W_in * s1
    bv1 = b_in * s1 + (bn1_b - bn1_m * s1)
    s2 = bn2_g / jnp.sqrt(bn2_v + 1e-5)
    t2 = bn2_b - bn2_m * s2
    s3 = bn3_g / jnp.sqrt(bn3_v + 1e-5)
    Wf1 = W_fc1 * s3
    bf1 = b_fc1 * s3 + (bn3_b - bn3_m * s3)
    W_ihT = W_ih.T
    W_hhT = W_hh.T

    npad = E_PAD - E
    src2 = jnp.concatenate(
        [edge_index[0], jnp.zeros((npad,), jnp.int32)])
    dst2 = jnp.concatenate(
        [edge_index[1], jnp.full((npad,), NP - 1, jnp.int32)]
    ).reshape(NC * NS, NCHUNK, CH)
    zeros = jnp.zeros((RPT, H), jnp.float32)

    xp, m = _dense0(x, W1, bv1[None], Wg[0])
    h = xp
    for i in range(STEPS):
        parts = _sc_scatter(src2, dst2, m, zeros).reshape(2, NP, H)
        if i < STEPS - 1:
            h, m = _gru_step(parts, h, W_ihT, W_hhT, b_ih[None], b_hh[None],
                             Wg[i + 1])
        else:
            zfeat = _gru_last(parts, h, W_ihT, W_hhT, b_ih[None], b_hh[None],
                              xp, s2[None], t2[None])
    return _pool_fc(zfeat, batch.reshape(N, 1), Wf1, bf1[None], W_fc2,
                    b_fc2[None])


# speedup vs baseline: 1.1283x; 1.1283x over previous
import jax
import jax.numpy as jnp
from jax import lax
from jax.experimental import pallas as pl
from jax.experimental.pallas import tpu as pltpu
from jax.experimental.pallas import tpu_sc as plsc

N = 10000
E = 320000
D = 128
H = 128
G = 64
STEPS = 4

NC = 2
NS = 16
CH = 128
NP = 10240
RPT = NP // NS
EPT = 10240
E_PAD = EPT * NC * NS
NCHUNK = EPT // CH


_sc_mesh = plsc.VectorSubcoreMesh(core_axis_name="c", subcore_axis_name="s")


def _sc_scatter_body(src_hbm, dst_hbm, m_hbm, zeros_hbm, out_hbm,
                     srcb, dstb, rows, agg, sem):
    cid = lax.axis_index("c")
    sid = lax.axis_index("s")
    tid = cid * NS + sid
    pltpu.sync_copy(src_hbm.at[tid], srcb)
    pltpu.sync_copy(dst_hbm.at[tid], dstb)
    pltpu.sync_copy(zeros_hbm, agg.at[pl.ds(sid * RPT, RPT)])
    plsc.subcore_barrier()

    def body(j, carry):
        pltpu.async_copy(m_hbm.at[srcb.at[j]], rows, sem).wait()
        pltpu.sync_copy(rows, agg.at[dstb.at[j]], add=True)
        return carry

    lax.fori_loop(0, NCHUNK, body, 0)
    plsc.subcore_barrier()
    pltpu.sync_copy(agg.at[pl.ds(sid * RPT, RPT)],
                    out_hbm.at[pl.ds(cid * NP + sid * RPT, RPT)])


_sc_scatter = pl.kernel(
    _sc_scatter_body,
    out_type=jax.ShapeDtypeStruct((2 * NP, H), jnp.float32),
    mesh=_sc_mesh,
    scratch_types=[
        pltpu.VMEM((NCHUNK, CH), jnp.int32),
        pltpu.VMEM((NCHUNK, CH), jnp.int32),
        pltpu.VMEM((CH, H), jnp.float32),
        pltpu.VMEM_SHARED((NP, H), jnp.float32),
        pltpu.SemaphoreType.DMA,
    ],
)


_BLK = 1000
_NBLK = N // _BLK


def _dense0_body(x_ref, w1_ref, b1_ref, wg0_ref, xp_ref, m0_ref):
    xp = jnp.dot(x_ref[...], w1_ref[...], preferred_element_type=jnp.float32)
    xp = jnp.maximum(xp + b1_ref[...], 0.0)
    xp_ref[...] = xp
    m0_ref[...] = jnp.dot(xp, wg0_ref[...], preferred_element_type=jnp.float32)


_dense0 = pl.pallas_call(
    _dense0_body,
    grid=(_NBLK,),
    in_specs=[
        pl.BlockSpec((_BLK, D), lambda i: (i, 0)),
        pl.BlockSpec((D, H), lambda i: (0, 0)),
        pl.BlockSpec((1, H), lambda i: (0, 0)),
        pl.BlockSpec((H, H), lambda i: (0, 0)),
    ],
    out_specs=[
        pl.BlockSpec((_BLK, H), lambda i: (i, 0)),
        pl.BlockSpec((_BLK, H), lambda i: (i, 0)),
    ],
    out_shape=[jax.ShapeDtypeStruct((N, H), jnp.float32),
               jax.ShapeDtypeStruct((N, H), jnp.float32)],
)


def _gru_core(agg, h, wih_ref, whh_ref, bih_ref, bhh_ref):
    gi = jnp.dot(agg, wih_ref[...], preferred_element_type=jnp.float32)
    gi = gi + bih_ref[...]
    gh = jnp.dot(h, whh_ref[...], preferred_element_type=jnp.float32)
    gh = gh + bhh_ref[...]
    r = jax.nn.sigmoid(gi[:, :H] + gh[:, :H])
    z = jax.nn.sigmoid(gi[:, H:2 * H] + gh[:, H:2 * H])
    n = jnp.tanh(gi[:, 2 * H:] + r * gh[:, 2 * H:])
    return (1.0 - z) * n + z * h


def _gru_step_body(agg_ref, h_ref, wih_ref, whh_ref, bih_ref, bhh_ref,
                   wgn_ref, h_out_ref, m_out_ref):
    agg = agg_ref[0] + agg_ref[1]
    hn = _gru_core(agg, h_ref[...], wih_ref, whh_ref, bih_ref, bhh_ref)
    h_out_ref[...] = hn
    m_out_ref[...] = jnp.dot(hn, wgn_ref[...], preferred_element_type=jnp.float32)


_gru_step = pl.pallas_call(
    _gru_step_body,
    grid=(_NBLK,),
    in_specs=[
        pl.BlockSpec((2, _BLK, H), lambda i: (0, i, 0)),
        pl.BlockSpec((_BLK, H), lambda i: (i, 0)),
        pl.BlockSpec((H, 3 * H), lambda i: (0, 0)),
        pl.BlockSpec((H, 3 * H), lambda i: (0, 0)),
        pl.BlockSpec((1, 3 * H), lambda i: (0, 0)),
        pl.BlockSpec((1, 3 * H), lambda i: (0, 0)),
        pl.BlockSpec((H, H), lambda i: (0, 0)),
    ],
    out_specs=[
        pl.BlockSpec((_BLK, H), lambda i: (i, 0)),
        pl.BlockSpec((_BLK, H), lambda i: (i, 0)),
    ],
    out_shape=[jax.ShapeDtypeStruct((N, H), jnp.float32),
               jax.ShapeDtypeStruct((N, H), jnp.float32)],
)


def _gru_last_body(agg_ref, h_ref, wih_ref, whh_ref, bih_ref, bhh_ref,
                   xp_ref, s2_ref, t2_ref, z_out_ref):
    agg = agg_ref[0] + agg_ref[1]
    hn = _gru_core(agg, h_ref[...], wih_ref, whh_ref, bih_ref, bhh_ref)
    z_out_ref[...] = jnp.maximum(
        xp_ref[...] + hn * s2_ref[...] + t2_ref[...], 0.0)


_gru_last = pl.pallas_call(
    _gru_last_body,
    grid=(_NBLK,),
    in_specs=[
        pl.BlockSpec((2, _BLK, H), lambda i: (0, i, 0)),
        pl.BlockSpec((_BLK, H), lambda i: (i, 0)),
        pl.BlockSpec((H, 3 * H), lambda i: (0, 0)),
        pl.BlockSpec((H, 3 * H), lambda i: (0, 0)),
        pl.BlockSpec((1, 3 * H), lambda i: (0, 0)),
        pl.BlockSpec((1, 3 * H), lambda i: (0, 0)),
        pl.BlockSpec((_BLK, H), lambda i: (i, 0)),
        pl.BlockSpec((1, H), lambda i: (0, 0)),
        pl.BlockSpec((1, H), lambda i: (0, 0)),
    ],
    out_specs=pl.BlockSpec((_BLK, H), lambda i: (i, 0)),
    out_shape=jax.ShapeDtypeStruct((N, H), jnp.float32),
)


def _pool_fc_body(z_ref, b_ref, wf1_ref, bf1_ref, wf2_ref, bf2_ref,
                  out_ref, pooled):
    zval = z_ref[...]
    bval = b_ref[...]
    neg = jnp.finfo(jnp.float32).min

    def body(g, carry):
        mg = jnp.where(bval == g, zval, neg)
        pooled[pl.ds(g, 1), :] = jnp.max(mg, axis=0, keepdims=True)
        return carry

    lax.fori_loop(0, G, body, 0)
    y = jnp.dot(pooled[...], wf1_ref[...], preferred_element_type=jnp.float32)
    y = jnp.maximum(y + bf1_ref[...], 0.0)
    out_ref[...] = jnp.dot(y, wf2_ref[...],
                           preferred_element_type=jnp.float32) + bf2_ref[...]


_pool_fc = pl.pallas_call(
    _pool_fc_body,
    in_specs=[
        pl.BlockSpec((N, H), lambda: (0, 0)),
        pl.BlockSpec((N, 1), lambda: (0, 0)),
        pl.BlockSpec((H, H // 2), lambda: (0, 0)),
        pl.BlockSpec((1, H // 2), lambda: (0, 0)),
        pl.BlockSpec((H // 2, 2), lambda: (0, 0)),
        pl.BlockSpec((1, 2), lambda: (0, 0)),
    ],
    out_specs=pl.BlockSpec((G, 2), lambda: (0, 0)),
    out_shape=jax.ShapeDtypeStruct((G, 2), jnp.float32),
    scratch_shapes=[pltpu.VMEM((G, H), jnp.float32)],
)


def kernel(x, edge_index, batch, W_in, b_in, bn1_g, bn1_b, bn1_m, bn1_v, Wg,
           W_ih, W_hh, b_ih, b_hh, bn2_g, bn2_b, bn2_m, bn2_v, W_fc1, b_fc1,
           bn3_g, bn3_b, bn3_m, bn3_v, W_fc2, b_fc2):
    s1 = bn1_g / jnp.sqrt(bn1_v + 1e-5)
    W1 = W_in * s1
    bv1 = b_in * s1 + (bn1_b - bn1_m * s1)
    s2 = bn2_g / jnp.sqrt(bn2_v + 1e-5)
    t2 = bn2_b - bn2_m * s2
    s3 = bn3_g / jnp.sqrt(bn3_v + 1e-5)
    Wf1 = W_fc1 * s3
    bf1 = b_fc1 * s3 + (bn3_b - bn3_m * s3)
    W_ihT = W_ih.T
    W_hhT = W_hh.T

    npad = E_PAD - E
    src2 = jnp.concatenate(
        [edge_index[0], jnp.zeros((npad,), jnp.int32)]
    ).reshape(NC * NS, NCHUNK, CH)
    dst2 = jnp.concatenate(
        [edge_index[1], jnp.full((npad,), NP - 1, jnp.int32)]
    ).reshape(NC * NS, NCHUNK, CH)
    zeros = jnp.zeros((RPT, H), jnp.float32)

    xp, m = _dense0(x, W1, bv1[None], Wg[0])
    h = xp
    for i in range(STEPS):
        parts = _sc_scatter(src2, dst2, m, zeros).reshape(2, NP, H)
        if i < STEPS - 1:
            h, m = _gru_step(parts, h, W_ihT, W_hhT, b_ih[None], b_hh[None],
                             Wg[i + 1])
        else:
            zfeat = _gru_last(parts, h, W_ihT, W_hhT, b_ih[None], b_hh[None],
                              xp, s2[None], t2[None])
    return _pool_fc(zfeat, batch.reshape(N, 1), Wf1, bf1[None], W_fc2,
                    b_fc2[None])


# speedup vs baseline: 3.2198x; 2.8537x over previous
import jax
import jax.numpy as jnp
from jax import lax
from jax.experimental import pallas as pl
from jax.experimental.pallas import tpu as pltpu
from jax.experimental.pallas import tpu_sc as plsc

N = 10000
E = 320000
D = 128
H = 128
G = 64
STEPS = 4

NC = 2
NS = 16
CH = 80
NP = 10240
RPT = NP // NS
EPT = E // (NC * NS)
NCHUNK = EPT // CH


_sc_mesh = plsc.VectorSubcoreMesh(core_axis_name="c", subcore_axis_name="s")


NRING = NCHUNK - 1


def _sc_scatter_body(src_hbm, dst_hbm, m_hbm, zeros_hbm, out_hbm,
                     srcb, dstb, rows, agg, gsems):
    cid = lax.axis_index("c")
    sid = lax.axis_index("s")
    tid = cid * NS + sid
    pltpu.sync_copy(src_hbm.at[pl.ds(tid * EPT, EPT)], srcb)
    pltpu.sync_copy(dst_hbm.at[tid], dstb)
    pltpu.sync_copy(zeros_hbm, agg.at[pl.ds(sid * RPT, RPT)])
    plsc.subcore_barrier()

    def gidx(j):
        return srcb.at[pl.ds(j * CH, CH)]

    for b in range(2):
        pltpu.async_copy(m_hbm.at[gidx(b)], rows.at[b], gsems.at[b])

    def body(i, carry):
        j0 = i * 2
        for b in range(2):
            j = j0 + b
            pltpu.make_async_copy(m_hbm.at[gidx(j)], rows.at[b],
                                  gsems.at[b]).wait()
            pltpu.sync_copy(rows.at[b], agg.at[dstb.at[j]], add=True)
            pltpu.async_copy(m_hbm.at[gidx((j + 2) % NRING)], rows.at[b],
                             gsems.at[b])
        return carry

    lax.fori_loop(0, NRING // 2, body, 0)
    for b in range(2):
        pltpu.make_async_copy(m_hbm.at[gidx(b)], rows.at[b],
                              gsems.at[b]).wait()
    pltpu.async_copy(m_hbm.at[gidx(NCHUNK - 1)], rows.at[0],
                     gsems.at[0]).wait()
    pltpu.sync_copy(rows.at[0], agg.at[dstb.at[NCHUNK - 1]], add=True)

    plsc.subcore_barrier()
    pltpu.sync_copy(agg.at[pl.ds(sid * RPT, RPT)],
                    out_hbm.at[pl.ds(cid * NP + sid * RPT, RPT)])


_sc_scatter = pl.kernel(
    _sc_scatter_body,
    out_type=jax.ShapeDtypeStruct((2 * NP, H), jnp.float32),
    mesh=_sc_mesh,
    scratch_types=[
        pltpu.VMEM((EPT,), jnp.int32),
        pltpu.VMEM((NCHUNK, CH), jnp.int32),
        pltpu.VMEM((2, CH, H), jnp.float32),
        pltpu.VMEM_SHARED((NP, H), jnp.float32),
        pltpu.SemaphoreType.DMA((2,)),
    ],
)


_BLK = 1000
_NBLK = N // _BLK


def _bn(x, g, b, m, v):
    return (x - m) / jnp.sqrt(v + 1e-5) * g + b


def _dot3(a, b):
    a_hi = a.astype(jnp.bfloat16)
    b_hi = b.astype(jnp.bfloat16)
    a_lo = (a - a_hi.astype(jnp.float32)).astype(jnp.bfloat16)
    b_lo = (b - b_hi.astype(jnp.float32)).astype(jnp.bfloat16)

    def d(u, v):
        return jnp.dot(u, v, preferred_element_type=jnp.float32)

    return d(a_lo, b_hi) + d(a_hi, b_lo) + d(a_hi, b_hi)


def _dense0_body(x_ref, w1_ref, b1_ref, g_ref, bb_ref, m_ref, v_ref,
                 wg0_ref, xp_ref, m0_ref):
    xp = jnp.dot(x_ref[...], w1_ref[...], preferred_element_type=jnp.float32)
    xp = jnp.maximum(
        _bn(xp + b1_ref[...], g_ref[...], bb_ref[...], m_ref[...],
            v_ref[...]), 0.0)
    xp_ref[...] = xp
    m0_ref[...] = jnp.dot(xp, wg0_ref[...], preferred_element_type=jnp.float32)


_dense0 = pl.pallas_call(
    _dense0_body,
    grid=(_NBLK,),
    in_specs=[
        pl.BlockSpec((_BLK, D), lambda i: (i, 0)),
        pl.BlockSpec((D, H), lambda i: (0, 0)),
        pl.BlockSpec((1, H), lambda i: (0, 0)),
        pl.BlockSpec((1, H), lambda i: (0, 0)),
        pl.BlockSpec((1, H), lambda i: (0, 0)),
        pl.BlockSpec((1, H), lambda i: (0, 0)),
        pl.BlockSpec((1, H), lambda i: (0, 0)),
        pl.BlockSpec((H, H), lambda i: (0, 0)),
    ],
    out_specs=[
        pl.BlockSpec((_BLK, H), lambda i: (i, 0)),
        pl.BlockSpec((_BLK, H), lambda i: (i, 0)),
    ],
    out_shape=[jax.ShapeDtypeStruct((N, H), jnp.float32),
               jax.ShapeDtypeStruct((N, H), jnp.float32)],
)


def _gru_core(agg, h, wih_ref, whh_ref, bih_ref, bhh_ref):
    gi = jnp.dot(agg, wih_ref[...], preferred_element_type=jnp.float32)
    gi = gi + bih_ref[...]
    gh = jnp.dot(h, whh_ref[...], preferred_element_type=jnp.float32)
    gh = gh + bhh_ref[...]
    r = jax.nn.sigmoid(gi[:, :H] + gh[:, :H])
    z = jax.nn.sigmoid(gi[:, H:2 * H] + gh[:, H:2 * H])
    n = jnp.tanh(gi[:, 2 * H:] + r * gh[:, 2 * H:])
    return (1.0 - z) * n + z * h


def _gru_step_body(agg_ref, h_ref, wih_ref, whh_ref, bih_ref, bhh_ref,
                   wgn_ref, h_out_ref, m_out_ref):
    agg = agg_ref[0] + agg_ref[1]
    hn = _gru_core(agg, h_ref[...], wih_ref, whh_ref, bih_ref, bhh_ref)
    h_out_ref[...] = hn
    m_out_ref[...] = jnp.dot(hn, wgn_ref[...], preferred_element_type=jnp.float32)


_gru_step = pl.pallas_call(
    _gru_step_body,
    grid=(_NBLK,),
    in_specs=[
        pl.BlockSpec((2, _BLK, H), lambda i: (0, i, 0)),
        pl.BlockSpec((_BLK, H), lambda i: (i, 0)),
        pl.BlockSpec((H, 3 * H), lambda i: (0, 0)),
        pl.BlockSpec((H, 3 * H), lambda i: (0, 0)),
        pl.BlockSpec((1, 3 * H), lambda i: (0, 0)),
        pl.BlockSpec((1, 3 * H), lambda i: (0, 0)),
        pl.BlockSpec((H, H), lambda i: (0, 0)),
    ],
    out_specs=[
        pl.BlockSpec((_BLK, H), lambda i: (i, 0)),
        pl.BlockSpec((_BLK, H), lambda i: (i, 0)),
    ],
    out_shape=[jax.ShapeDtypeStruct((N, H), jnp.float32),
               jax.ShapeDtypeStruct((N, H), jnp.float32)],
)


def _gru_last_body(agg_ref, h_ref, wih_ref, whh_ref, bih_ref, bhh_ref,
                   xp_ref, g_ref, bb_ref, m_ref, v_ref, z_out_ref):
    agg = agg_ref[0] + agg_ref[1]
    hn = _gru_core(agg, h_ref[...], wih_ref, whh_ref, bih_ref, bhh_ref)
    hg = _bn(hn, g_ref[...], bb_ref[...], m_ref[...], v_ref[...])
    z_out_ref[...] = jnp.maximum(xp_ref[...] + hg, 0.0)


_gru_last = pl.pallas_call(
    _gru_last_body,
    grid=(_NBLK,),
    in_specs=[
        pl.BlockSpec((2, _BLK, H), lambda i: (0, i, 0)),
        pl.BlockSpec((_BLK, H), lambda i: (i, 0)),
        pl.BlockSpec((H, 3 * H), lambda i: (0, 0)),
        pl.BlockSpec((H, 3 * H), lambda i: (0, 0)),
        pl.BlockSpec((1, 3 * H), lambda i: (0, 0)),
        pl.BlockSpec((1, 3 * H), lambda i: (0, 0)),
        pl.BlockSpec((_BLK, H), lambda i: (i, 0)),
        pl.BlockSpec((1, H), lambda i: (0, 0)),
        pl.BlockSpec((1, H), lambda i: (0, 0)),
        pl.BlockSpec((1, H), lambda i: (0, 0)),
        pl.BlockSpec((1, H), lambda i: (0, 0)),
    ],
    out_specs=pl.BlockSpec((_BLK, H), lambda i: (i, 0)),
    out_shape=jax.ShapeDtypeStruct((N, H), jnp.float32),
)


def _pool_fc_body(z_ref, b_ref, wf1_ref, bf1_ref, g_ref, bb_ref, m_ref,
                  v_ref, wf2_ref, bf2_ref, out_ref, pooled):
    zval = z_ref[...]
    bval = b_ref[...]
    neg = jnp.finfo(jnp.float32).min

    def body(g, carry):
        mg = jnp.where(bval == g, zval, neg)
        pooled[pl.ds(g, 1), :] = jnp.max(mg, axis=0, keepdims=True)
        return carry

    lax.fori_loop(0, G, body, 0)
    y = jnp.dot(pooled[...], wf1_ref[...], preferred_element_type=jnp.float32)
    y = jnp.maximum(
        _bn(y + bf1_ref[...], g_ref[...], bb_ref[...], m_ref[...],
            v_ref[...]), 0.0)
    out_ref[...] = jnp.dot(y, wf2_ref[...], preferred_element_type=jnp.float32) + bf2_ref[...]


_pool_fc = pl.pallas_call(
    _pool_fc_body,
    in_specs=[
        pl.BlockSpec((N, H), lambda: (0, 0)),
        pl.BlockSpec((N, 1), lambda: (0, 0)),
        pl.BlockSpec((H, H // 2), lambda: (0, 0)),
        pl.BlockSpec((1, H // 2), lambda: (0, 0)),
        pl.BlockSpec((1, H // 2), lambda: (0, 0)),
        pl.BlockSpec((1, H // 2), lambda: (0, 0)),
        pl.BlockSpec((1, H // 2), lambda: (0, 0)),
        pl.BlockSpec((1, H // 2), lambda: (0, 0)),
        pl.BlockSpec((H // 2, 2), lambda: (0, 0)),
        pl.BlockSpec((1, 2), lambda: (0, 0)),
    ],
    out_specs=pl.BlockSpec((G, 2), lambda: (0, 0)),
    out_shape=jax.ShapeDtypeStruct((G, 2), jnp.float32),
    scratch_shapes=[pltpu.VMEM((G, H), jnp.float32)],
)


def kernel(x, edge_index, batch, W_in, b_in, bn1_g, bn1_b, bn1_m, bn1_v, Wg,
           W_ih, W_hh, b_ih, b_hh, bn2_g, bn2_b, bn2_m, bn2_v, W_fc1, b_fc1,
           bn3_g, bn3_b, bn3_m, bn3_v, W_fc2, b_fc2):
    W_ihT = W_ih.T
    W_hhT = W_hh.T

    src2 = edge_index[0]
    dst2 = edge_index[1].reshape(NC * NS, NCHUNK, CH)
    zeros = jnp.zeros((RPT, H), jnp.float32)

    xp, m = _dense0(x, W_in, b_in[None], bn1_g[None], bn1_b[None],
                    bn1_m[None], bn1_v[None], Wg[0])
    h = xp
    for i in range(STEPS):
        parts = _sc_scatter(src2, dst2, m, zeros).reshape(2, NP, H)
        if i < STEPS - 1:
            h, m = _gru_step(parts, h, W_ihT, W_hhT, b_ih[None], b_hh[None],
                             Wg[i + 1])
        else:
            zfeat = _gru_last(parts, h, W_ihT, W_hhT, b_ih[None], b_hh[None],
                              xp, bn2_g[None], bn2_b[None], bn2_m[None],
                              bn2_v[None])
    return _pool_fc(zfeat, batch.reshape(N, 1), W_fc1, b_fc1[None],
                    bn3_g[None], bn3_b[None], bn3_m[None], bn3_v[None],
                    W_fc2, b_fc2[None])


# speedup vs baseline: 3.6900x; 1.1461x over previous
import jax
import jax.numpy as jnp
from jax import lax
from jax.experimental import pallas as pl
from jax.experimental.pallas import tpu as pltpu
from jax.experimental.pallas import tpu_sc as plsc

N = 10000
E = 320000
D = 128
H = 128
G = 64
STEPS = 4

NC = 2
NS = 16
CH = 80
NP = 10240
RPT = NP // NS
EPT = E // (NC * NS)
NCHUNK = EPT // CH


_sc_mesh = plsc.VectorSubcoreMesh(core_axis_name="c", subcore_axis_name="s")


RZT = 10
RZROWS = N // RZT


def _sc_scatter_body(src_hbm, dst_hbm, m_hbm, zeros_hbm, out_hbm,
                     srcb, dstb, rows, agg, gsems, ssems):
    cid = lax.axis_index("c")
    sid = lax.axis_index("s")
    tid = cid * NS + sid
    pltpu.sync_copy(src_hbm.at[pl.ds(tid * EPT, EPT)], srcb)
    pltpu.sync_copy(dst_hbm.at[pl.ds(tid * EPT, EPT)], dstb)

    @pl.when(sid < RZT)
    def _():
        pltpu.sync_copy(zeros_hbm, agg.at[pl.ds(sid * RZROWS, RZROWS)])

    plsc.subcore_barrier()

    def gidx(j):
        return srcb.at[pl.ds(j * CH, CH)]

    def didx(j):
        return dstb.at[pl.ds(j * CH, CH)]

    def g_start(j, b):
        pltpu.async_copy(m_hbm.at[gidx(j)], rows.at[b], gsems.at[b])

    def g_wait(j, b):
        pltpu.make_async_copy(m_hbm.at[gidx(j)], rows.at[b],
                              gsems.at[b]).wait()

    def s_start(j, b):
        pltpu.async_copy(rows.at[b], agg.at[didx(j)], ssems.at[b], add=True)

    def s_wait(j, b):
        pltpu.make_async_copy(rows.at[b], agg.at[didx(j)],
                              ssems.at[b]).wait()

    g_start(0, 0)
    g_start(1, 1)
    g_wait(0, 0)
    s_start(0, 0)
    g_start(2, 2)
    g_wait(1, 1)
    s_start(1, 1)
    s_wait(0, 0)
    g_start(3, 0)

    def body(i, carry):
        j0 = 3 * i + 2
        for k in range(3):
            j = j0 + k
            b = (2 + k) % 3
            bp = (1 + k) % 3
            g_wait(j, b)
            s_start(j, b)
            s_wait(j - 1, bp)
            g_start(j + 2, bp)
        return carry

    lax.fori_loop(0, (NCHUNK - 5) // 3, body, 0)

    g_wait(NCHUNK - 3, 2)
    s_start(NCHUNK - 3, 2)
    s_wait(NCHUNK - 4, 1)
    g_start(NCHUNK - 1, 1)
    g_wait(NCHUNK - 2, 0)
    s_start(NCHUNK - 2, 0)
    s_wait(NCHUNK - 3, 2)
    g_wait(NCHUNK - 1, 1)
    s_start(NCHUNK - 1, 1)
    s_wait(NCHUNK - 2, 0)
    s_wait(NCHUNK - 1, 1)

    plsc.subcore_barrier()

    @pl.when(sid < RZT)
    def _():
        pltpu.sync_copy(agg.at[pl.ds(sid * RZROWS, RZROWS)],
                        out_hbm.at[pl.ds(cid * N + sid * RZROWS, RZROWS)])


_sc_scatter = pl.kernel(
    _sc_scatter_body,
    out_type=jax.ShapeDtypeStruct((2 * N, H), jnp.float32),
    mesh=_sc_mesh,
    scratch_types=[
        pltpu.VMEM((EPT,), jnp.int32),
        pltpu.VMEM((EPT,), jnp.int32),
        pltpu.VMEM((3, CH, H), jnp.float32),
        pltpu.VMEM_SHARED((N, H), jnp.float32),
        pltpu.SemaphoreType.DMA((3,)),
        pltpu.SemaphoreType.DMA((3,)),
    ],
)


_BLK = 1000
_NBLK = N // _BLK


def _bn(x, g, b, m, v):
    return (x - m) / jnp.sqrt(v + 1e-5) * g + b


def _dot3(a, b):
    a_hi = a.astype(jnp.bfloat16)
    b_hi = b.astype(jnp.bfloat16)
    a_lo = (a - a_hi.astype(jnp.float32)).astype(jnp.bfloat16)
    b_lo = (b - b_hi.astype(jnp.float32)).astype(jnp.bfloat16)

    def d(u, v):
        return jnp.dot(u, v, preferred_element_type=jnp.float32)

    return d(a_lo, b_hi) + d(a_hi, b_lo) + d(a_hi, b_hi)


def _dense0_body(x_ref, w1_ref, b1_ref, g_ref, bb_ref, m_ref, v_ref,
                 wg0_ref, xp_ref, m0_ref):
    xp = jnp.dot(x_ref[...], w1_ref[...], preferred_element_type=jnp.float32)
    xp = jnp.maximum(
        _bn(xp + b1_ref[...], g_ref[...], bb_ref[...], m_ref[...],
            v_ref[...]), 0.0)
    xp_ref[...] = xp
    m0_ref[...] = jnp.dot(xp, wg0_ref[...], preferred_element_type=jnp.float32)


_dense0 = pl.pallas_call(
    _dense0_body,
    grid=(_NBLK,),
    in_specs=[
        pl.BlockSpec((_BLK, D), lambda i: (i, 0)),
        pl.BlockSpec((D, H), lambda i: (0, 0)),
        pl.BlockSpec((1, H), lambda i: (0, 0)),
        pl.BlockSpec((1, H), lambda i: (0, 0)),
        pl.BlockSpec((1, H), lambda i: (0, 0)),
        pl.BlockSpec((1, H), lambda i: (0, 0)),
        pl.BlockSpec((1, H), lambda i: (0, 0)),
        pl.BlockSpec((H, H), lambda i: (0, 0)),
    ],
    out_specs=[
        pl.BlockSpec((_BLK, H), lambda i: (i, 0)),
        pl.BlockSpec((_BLK, H), lambda i: (i, 0)),
    ],
    out_shape=[jax.ShapeDtypeStruct((N, H), jnp.float32),
               jax.ShapeDtypeStruct((N, H), jnp.float32)],
)


def _gru_core(agg, h, wih_ref, whh_ref, bih_ref, bhh_ref):
    gi = jnp.dot(agg, wih_ref[...], preferred_element_type=jnp.float32)
    gi = gi + bih_ref[...]
    gh = jnp.dot(h, whh_ref[...], preferred_element_type=jnp.float32)
    gh = gh + bhh_ref[...]
    r = jax.nn.sigmoid(gi[:, :H] + gh[:, :H])
    z = jax.nn.sigmoid(gi[:, H:2 * H] + gh[:, H:2 * H])
    n = jnp.tanh(gi[:, 2 * H:] + r * gh[:, 2 * H:])
    return (1.0 - z) * n + z * h


def _gru_step_body(agg_ref, h_ref, wih_ref, whh_ref, bih_ref, bhh_ref,
                   wgn_ref, h_out_ref, m_out_ref):
    agg = agg_ref[0] + agg_ref[1]
    hn = _gru_core(agg, h_ref[...], wih_ref, whh_ref, bih_ref, bhh_ref)
    h_out_ref[...] = hn
    m_out_ref[...] = jnp.dot(hn, wgn_ref[...], preferred_element_type=jnp.float32)


_gru_step = pl.pallas_call(
    _gru_step_body,
    grid=(_NBLK,),
    in_specs=[
        pl.BlockSpec((2, _BLK, H), lambda i: (0, i, 0)),
        pl.BlockSpec((_BLK, H), lambda i: (i, 0)),
        pl.BlockSpec((H, 3 * H), lambda i: (0, 0)),
        pl.BlockSpec((H, 3 * H), lambda i: (0, 0)),
        pl.BlockSpec((1, 3 * H), lambda i: (0, 0)),
        pl.BlockSpec((1, 3 * H), lambda i: (0, 0)),
        pl.BlockSpec((H, H), lambda i: (0, 0)),
    ],
    out_specs=[
        pl.BlockSpec((_BLK, H), lambda i: (i, 0)),
        pl.BlockSpec((_BLK, H), lambda i: (i, 0)),
    ],
    out_shape=[jax.ShapeDtypeStruct((N, H), jnp.float32),
               jax.ShapeDtypeStruct((N, H), jnp.float32)],
)


def _gru_last_body(agg_ref, h_ref, wih_ref, whh_ref, bih_ref, bhh_ref,
                   xp_ref, g_ref, bb_ref, m_ref, v_ref, z_out_ref):
    agg = agg_ref[0] + agg_ref[1]
    hn = _gru_core(agg, h_ref[...], wih_ref, whh_ref, bih_ref, bhh_ref)
    hg = _bn(hn, g_ref[...], bb_ref[...], m_ref[...], v_ref[...])
    z_out_ref[...] = jnp.maximum(xp_ref[...] + hg, 0.0)


_gru_last = pl.pallas_call(
    _gru_last_body,
    grid=(_NBLK,),
    in_specs=[
        pl.BlockSpec((2, _BLK, H), lambda i: (0, i, 0)),
        pl.BlockSpec((_BLK, H), lambda i: (i, 0)),
        pl.BlockSpec((H, 3 * H), lambda i: (0, 0)),
        pl.BlockSpec((H, 3 * H), lambda i: (0, 0)),
        pl.BlockSpec((1, 3 * H), lambda i: (0, 0)),
        pl.BlockSpec((1, 3 * H), lambda i: (0, 0)),
        pl.BlockSpec((_BLK, H), lambda i: (i, 0)),
        pl.BlockSpec((1, H), lambda i: (0, 0)),
        pl.BlockSpec((1, H), lambda i: (0, 0)),
        pl.BlockSpec((1, H), lambda i: (0, 0)),
        pl.BlockSpec((1, H), lambda i: (0, 0)),
    ],
    out_specs=pl.BlockSpec((_BLK, H), lambda i: (i, 0)),
    out_shape=jax.ShapeDtypeStruct((N, H), jnp.float32),
)


def _pool_fc_body(z_ref, b_ref, wf1_ref, bf1_ref, g_ref, bb_ref, m_ref,
                  v_ref, wf2_ref, bf2_ref, out_ref, pooled):
    zval = z_ref[...]
    bval = b_ref[...]
    neg = jnp.finfo(jnp.float32).min

    def body(g, carry):
        mg = jnp.where(bval == g, zval, neg)
        pooled[pl.ds(g, 1), :] = jnp.max(mg, axis=0, keepdims=True)
        return carry

    lax.fori_loop(0, G, body, 0)
    y = jnp.dot(pooled[...], wf1_ref[...], preferred_element_type=jnp.float32)
    y = jnp.maximum(
        _bn(y + bf1_ref[...], g_ref[...], bb_ref[...], m_ref[...],
            v_ref[...]), 0.0)
    out_ref[...] = jnp.dot(y, wf2_ref[...], preferred_element_type=jnp.float32) + bf2_ref[...]


_pool_fc = pl.pallas_call(
    _pool_fc_body,
    in_specs=[
        pl.BlockSpec((N, H), lambda: (0, 0)),
        pl.BlockSpec((N, 1), lambda: (0, 0)),
        pl.BlockSpec((H, H // 2), lambda: (0, 0)),
        pl.BlockSpec((1, H // 2), lambda: (0, 0)),
        pl.BlockSpec((1, H // 2), lambda: (0, 0)),
        pl.BlockSpec((1, H // 2), lambda: (0, 0)),
        pl.BlockSpec((1, H // 2), lambda: (0, 0)),
        pl.BlockSpec((1, H // 2), lambda: (0, 0)),
        pl.BlockSpec((H // 2, 2), lambda: (0, 0)),
        pl.BlockSpec((1, 2), lambda: (0, 0)),
    ],
    out_specs=pl.BlockSpec((G, 2), lambda: (0, 0)),
    out_shape=jax.ShapeDtypeStruct((G, 2), jnp.float32),
    scratch_shapes=[pltpu.VMEM((G, H), jnp.float32)],
)


def kernel(x, edge_index, batch, W_in, b_in, bn1_g, bn1_b, bn1_m, bn1_v, Wg,
           W_ih, W_hh, b_ih, b_hh, bn2_g, bn2_b, bn2_m, bn2_v, W_fc1, b_fc1,
           bn3_g, bn3_b, bn3_m, bn3_v, W_fc2, b_fc2):
    W_ihT = W_ih.T
    W_hhT = W_hh.T

    src2 = edge_index[0]
    dst2 = edge_index[1]
    zeros = jnp.zeros((RZROWS, H), jnp.float32)

    xp, m = _dense0(x, W_in, b_in[None], bn1_g[None], bn1_b[None],
                    bn1_m[None], bn1_v[None], Wg[0])
    h = xp
    for i in range(STEPS):
        parts = _sc_scatter(src2, dst2, m, zeros).reshape(2, N, H)
        if i < STEPS - 1:
            h, m = _gru_step(parts, h, W_ihT, W_hhT, b_ih[None], b_hh[None],
                             Wg[i + 1])
        else:
            zfeat = _gru_last(parts, h, W_ihT, W_hhT, b_ih[None], b_hh[None],
                              xp, bn2_g[None], bn2_b[None], bn2_m[None],
                              bn2_v[None])
    return _pool_fc(zfeat, batch.reshape(N, 1), W_fc1, b_fc1[None],
                    bn3_g[None], bn3_b[None], bn3_m[None], bn3_v[None],
                    W_fc2, b_fc2[None])
